# Initial kernel scaffold; baseline (speedup 1.0000x reference)
import functools

import jax
import jax.numpy as jnp
from jax import lax
from jax.experimental import pallas as pl
from jax.experimental.pallas import tpu as pltpu
from jax.experimental.pallas import tpu_sc as plsc

L = 512
D = 256
H = 8
DH = D // H
VP = 64
AV = 64
NEG = -1e30


def _tc_body(se_ref, vx_ref, ttr_ref, admr_ref, ttc_ref, admc_ref,
             wocc_ref, wvis_ref, wnxt_ref, a0_ref, a1_ref, r_ref,
             tv_ref, wcls_ref, bcls_ref, out_ref, log_ref):
    f32 = jnp.float32
    se = se_ref[0]
    vx = vx_ref[0]
    ttr = ttr_ref[0]
    admr = admr_ref[0]
    ttc = ttc_ref[0]
    admc = admc_ref[0]

    keep_r = (ttr != 5) & (admr != 0)
    occ_r = keep_r & (ttr == 1)
    keep_c = (ttc != 5) & (admc != 0)
    occ_c = keep_c & (ttc == 1)
    occ_cf = occ_c.astype(f32)

    rows_a = lax.broadcasted_iota(jnp.int32, (AV, L), 0)
    oh_r = ((admr == rows_a) & keep_r).astype(f32)
    p_col = jnp.max(oh_r, axis=1, keepdims=True)
    lt_i = lax.broadcasted_iota(jnp.int32, (AV, AV), 0)
    lt_j = lax.broadcasted_iota(jnp.int32, (AV, AV), 1)
    ltri = (lt_j < lt_i).astype(f32)
    rank_col = jnp.dot(ltri, p_col, preferred_element_type=f32)
    nv = jnp.sum(p_col)

    cols_a = lax.broadcasted_iota(jnp.int32, (L, AV), 1)
    oh_c = ((admc == cols_a) & keep_c).astype(f32)
    utri = (lt_i < lt_j).astype(f32)
    p_row = jnp.max(oh_c, axis=0, keepdims=True)
    rank_row = jnp.dot(p_row, utri, preferred_element_type=f32)

    dst_r = jnp.sum(oh_r * rank_col, axis=0, keepdims=True)
    dst_c = jnp.sum(oh_c * rank_row, axis=1, keepdims=True)

    rows_v = lax.broadcasted_iota(jnp.int32, (VP, L), 0).astype(f32)
    cols_v = lax.broadcasted_iota(jnp.int32, (L, VP), 1).astype(f32)
    mf_b = (dst_r == rows_v) & occ_r
    mft_b = (dst_c == cols_v) & occ_c
    mf = mf_b.astype(f32)
    mft = mft_b.astype(f32)

    h_occ = jnp.dot(se, wocc_ref[...], preferred_element_type=f32)
    h_vis = jnp.dot(vx, wvis_ref[...], preferred_element_type=f32)
    h_nxt = jnp.dot(vx, wnxt_ref[...], preferred_element_type=f32)

    e_src = jnp.dot(h_occ, a0_ref[...], preferred_element_type=f32)
    e_dst = jnp.dot(h_vis, a1_ref[...], preferred_element_type=f32)
    ge = jnp.dot(mft, e_dst, preferred_element_type=f32)
    e = e_src + ge
    e = jnp.where(e > 0, e, 0.2 * e)

    head_i = lax.broadcasted_iota(jnp.int32, (H, 1), 0)
    mu_cols = []
    for h in range(H):
        u_h = (head_i == h).astype(f32)
        e_h = jnp.dot(e, u_h, preferred_element_type=f32)
        masked = jnp.where(mft_b, e_h, NEG)
        m_h = jnp.max(masked, axis=0, keepdims=True)
        m_h = jnp.where(m_h > -1e29, m_h, 0.0)
        mu_cols.append(jnp.sum(mft * m_h, axis=1, keepdims=True))
    m_used = jnp.concatenate(mu_cols, axis=1)

    ex = jnp.exp(e - m_used) * occ_cf
    den = jnp.dot(mf, ex, preferred_element_type=f32)
    rm = r_ref[...]
    ex_rep = jnp.dot(ex, rm, preferred_element_type=f32)
    num = jnp.dot(mf, ex_rep * h_occ, preferred_element_type=f32)
    den_rep = jnp.dot(den, rm, preferred_element_type=f32)
    agg1 = num / jnp.maximum(den_rep, 1e-9)

    s_i = lax.broadcasted_iota(jnp.int32, (VP, VP), 0)
    s_j = lax.broadcasted_iota(jnp.int32, (VP, VP), 1)
    shift = ((s_j == s_i - 1) & (s_i.astype(f32) < nv)).astype(f32)
    agg2 = jnp.dot(shift, h_nxt, preferred_element_type=f32)

    pre = agg1 + agg2 + h_vis
    out = jnp.where(pre > 0, pre, jnp.exp(pre) - 1.0)
    rowmask = lax.broadcasted_iota(jnp.int32, (VP, D), 0).astype(f32) < nv
    out_ref[0] = jnp.where(rowmask, out, 0.0)

    logit = jnp.sum(tv_ref[...] * wcls_ref[...], axis=1, keepdims=True)
    log_ref[0] = logit + bcls_ref[...]


def _tc_call(se3, vx3, ttr, admr, ttc, admc, w_occ, w_vis, w_next,
             a0, a1, rmat, tv, wcls, bcls, interpret=False):
    b = se3.shape[0]
    f32 = jnp.float32
    fixed = lambda *s: pl.BlockSpec(s, lambda i: (0,) * len(s))
    per_b = lambda *s: pl.BlockSpec(s, lambda i: (i,) + (0,) * (len(s) - 1))
    return pl.pallas_call(
        _tc_body,
        grid=(b,),
        in_specs=[
            per_b(1, L, D), per_b(1, VP, D),
            per_b(1, 1, L), per_b(1, 1, L),
            per_b(1, L, 1), per_b(1, L, 1),
            fixed(D, D), fixed(D, D), fixed(D, D),
            fixed(D, H), fixed(D, H), fixed(H, D),
            fixed(1, D), fixed(1, D), fixed(1, 1),
        ],
        out_specs=[per_b(1, VP, D), per_b(1, 1, 1)],
        out_shape=[
            jax.ShapeDtypeStruct((b, VP, D), f32),
            jax.ShapeDtypeStruct((b, 1, 1), f32),
        ],
        compiler_params=pltpu.CompilerParams(
            dimension_semantics=("arbitrary",),
        ),
        interpret=interpret,
    )(se3, vx3, ttr, admr, ttc, admc, w_occ, w_vis, w_next,
      a0, a1, rmat, tv, wcls, bcls)


def _sc_gather(table, idx_all):
    info = plsc.get_sparse_core_info()
    nw = info.num_cores * info.num_subcores
    n = idx_all.shape[0]
    per_w = n // nw
    ch = 128
    n_ch = per_w // ch
    mesh = plsc.VectorSubcoreMesh(core_axis_name="c", subcore_axis_name="s")

    @functools.partial(
        pl.kernel, mesh=mesh,
        out_type=jax.ShapeDtypeStruct((n, D), jnp.float32),
        scratch_types=[
            pltpu.VMEM((ch,), jnp.int32),
            pltpu.VMEM((ch, D), jnp.float32),
            pltpu.SemaphoreType.DMA,
        ],
    )
    def k(table_hbm, idx_hbm, out_hbm, idx_v, rows_v, sem):
        wid = lax.axis_index("s") * info.num_cores + lax.axis_index("c")
        base = wid * per_w
        for c in range(n_ch):
            off = base + c * ch
            pltpu.sync_copy(idx_hbm.at[pl.ds(off, ch)], idx_v)
            pltpu.async_copy(table_hbm.at[idx_v], rows_v, sem).wait()
            pltpu.sync_copy(rows_v, out_hbm.at[pl.ds(off, ch)])

    return k(table, idx_all)


def kernel(input_ids, token_types, adm_index, age_ids, diag_code_group_dicts,
           task_id, token_emb, task_emb_table, W_occ, W_vis, W_next,
           a_o2v, a_next, W_cls, b_cls):
    f32 = jnp.float32
    b = input_ids.shape[0]
    v = age_ids.shape[1]

    age_pad = jnp.concatenate(
        [age_ids.astype(jnp.int32), jnp.zeros((b, VP - v), jnp.int32)], axis=1)
    idx_all = jnp.concatenate(
        [input_ids.reshape(-1).astype(jnp.int32), age_pad.reshape(-1)])
    rows = _sc_gather(token_emb, idx_all)
    se3 = rows[:b * L].reshape(b, L, D)
    vx3 = rows[b * L:].reshape(b, VP, D)

    ttr = token_types.astype(jnp.int32).reshape(b, 1, L)
    admr = adm_index.astype(jnp.int32).reshape(b, 1, L)
    ttc = token_types.astype(jnp.int32).reshape(b, L, 1)
    admc = adm_index.astype(jnp.int32).reshape(b, L, 1)

    eye = jnp.eye(H, dtype=f32)
    a0 = (a_o2v[0][:, :, None] * eye[:, None, :]).reshape(D, H)
    a1 = (a_o2v[1][:, :, None] * eye[:, None, :]).reshape(D, H)
    rmat = jnp.repeat(eye, DH, axis=1)
    tv = jnp.take(task_emb_table, jnp.asarray(task_id, jnp.int32),
                  axis=0).reshape(1, D)
    wcls = W_cls.reshape(1, D)
    bcls = b_cls.reshape(1, 1)

    out_p, log3 = _tc_call(se3, vx3, ttr, admr, ttc, admc,
                           W_occ, W_vis, W_next, a0, a1, rmat, tv, wcls, bcls)
    return log3.reshape(b), out_p[:, :v, :]


# speedup vs baseline: 12.5750x; 12.5750x over previous
import functools

import jax
import jax.numpy as jnp
from jax import lax
from jax.experimental import pallas as pl
from jax.experimental.pallas import tpu as pltpu
from jax.experimental.pallas import tpu_sc as plsc

L = 512
D = 256
H = 8
DH = D // H
VP = 64
AV = 64
NEG = -1e30
_dot = functools.partial(jnp.dot, preferred_element_type=jnp.float32,
                         precision=jax.lax.Precision.HIGHEST)


def _dot_bf(x, w):
    return jnp.dot(x.astype(jnp.bfloat16), w.astype(jnp.bfloat16),
                   preferred_element_type=jnp.float32)


def _tc_body(se_ref, vx_ref, ttr_ref, admr_ref, ttc_ref, admc_ref,
             wocc_ref, wvis_ref, wnxt_ref, a0_ref, a1_ref, r_ref,
             tv_ref, wcls_ref, bcls_ref, out_ref, log_ref):
    f32 = jnp.float32
    se = se_ref[0]
    vx = vx_ref[0]
    ttr = ttr_ref[0]
    admr = admr_ref[0]
    ttc = ttc_ref[0]
    admc = admc_ref[0]

    keep_r = (ttr != 5) & (admr != 0)
    occ_r = keep_r & (ttr == 1)
    keep_c = (ttc != 5) & (admc != 0)
    occ_c = keep_c & (ttc == 1)
    occ_cf = occ_c.astype(f32)

    rows_a = lax.broadcasted_iota(jnp.int32, (AV, L), 0)
    oh_r = ((admr == rows_a) & keep_r).astype(f32)
    p_col = jnp.max(oh_r, axis=1, keepdims=True)
    lt_i = lax.broadcasted_iota(jnp.int32, (AV, AV), 0)
    lt_j = lax.broadcasted_iota(jnp.int32, (AV, AV), 1)
    ltri = (lt_j < lt_i).astype(f32)
    rank_col = _dot(ltri, p_col)
    nv = jnp.sum(p_col)

    cols_a = lax.broadcasted_iota(jnp.int32, (L, AV), 1)
    oh_c = ((admc == cols_a) & keep_c).astype(f32)
    utri = (lt_i < lt_j).astype(f32)
    p_row = jnp.max(oh_c, axis=0, keepdims=True)
    rank_row = _dot(p_row, utri)

    dst_r = jnp.sum(oh_r * rank_col, axis=0, keepdims=True)
    dst_c = jnp.sum(oh_c * rank_row, axis=1, keepdims=True)

    rows_v = lax.broadcasted_iota(jnp.int32, (VP, L), 0).astype(f32)
    cols_v = lax.broadcasted_iota(jnp.int32, (L, VP), 1).astype(f32)
    mf_b = (dst_r == rows_v) & occ_r
    mft_b = (dst_c == cols_v) & occ_c
    mf = mf_b.astype(f32)
    mft = mft_b.astype(f32)

    h_occ = _dot_bf(se, wocc_ref[...])
    h_vis = _dot_bf(vx, wvis_ref[...])
    h_nxt = _dot_bf(vx, wnxt_ref[...])

    e_src = _dot(h_occ, a0_ref[...])
    e_dst = _dot(h_vis, a1_ref[...])
    ge = _dot(mft, e_dst)
    e = e_src + ge
    e = jnp.where(e > 0, e, 0.2 * e)

    head_i = lax.broadcasted_iota(jnp.int32, (H, 1), 0)
    mu_cols = []
    for h in range(H):
        u_h = (head_i == h).astype(f32)
        e_h = _dot(e, u_h)
        masked = jnp.where(mft_b, e_h, NEG)
        m_h = jnp.max(masked, axis=0, keepdims=True)
        m_h = jnp.where(m_h > -1e29, m_h, 0.0)
        mu_cols.append(jnp.sum(mft * m_h, axis=1, keepdims=True))
    m_used = jnp.concatenate(mu_cols, axis=1)

    ex = jnp.exp(e - m_used) * occ_cf
    den = _dot(mf, ex)
    rm = r_ref[...]
    ex_rep = _dot(ex, rm)
    num = _dot(mf, ex_rep * h_occ)
    den_rep = _dot(den, rm)
    agg1 = num / jnp.maximum(den_rep, 1e-9)

    s_i = lax.broadcasted_iota(jnp.int32, (VP, VP), 0)
    s_j = lax.broadcasted_iota(jnp.int32, (VP, VP), 1)
    shift = ((s_j == s_i - 1) & (s_i.astype(f32) < nv)).astype(f32)
    agg2 = _dot(shift, h_nxt)

    pre = agg1 + agg2 + h_vis
    out = jnp.where(pre > 0, pre, jnp.exp(pre) - 1.0)
    rowmask = lax.broadcasted_iota(jnp.int32, (VP, D), 0).astype(f32) < nv
    out_ref[0] = jnp.where(rowmask, out, 0.0)

    logit = jnp.sum(tv_ref[...] * wcls_ref[...], axis=1, keepdims=True)
    log_ref[0] = logit + bcls_ref[...]


def _tc_call(se3, vx3, ttr, admr, ttc, admc, w_occ, w_vis, w_next,
             a0, a1, rmat, tv, wcls, bcls, interpret=False):
    b = se3.shape[0]
    f32 = jnp.float32
    fixed = lambda *s: pl.BlockSpec(s, lambda i: (0,) * len(s))
    per_b = lambda *s: pl.BlockSpec(s, lambda i: (i,) + (0,) * (len(s) - 1))
    return pl.pallas_call(
        _tc_body,
        grid=(b,),
        in_specs=[
            per_b(1, L, D), per_b(1, VP, D),
            per_b(1, 1, L), per_b(1, 1, L),
            per_b(1, L, 1), per_b(1, L, 1),
            fixed(D, D), fixed(D, D), fixed(D, D),
            fixed(D, H), fixed(D, H), fixed(H, D),
            fixed(1, D), fixed(1, D), fixed(1, 1),
        ],
        out_specs=[per_b(1, VP, D), per_b(1, 1, 1)],
        out_shape=[
            jax.ShapeDtypeStruct((b, VP, D), f32),
            jax.ShapeDtypeStruct((b, 1, 1), f32),
        ],
        compiler_params=pltpu.CompilerParams(
            dimension_semantics=("arbitrary",),
        ),
        interpret=interpret,
    )(se3, vx3, ttr, admr, ttc, admc, w_occ, w_vis, w_next,
      a0, a1, rmat, tv, wcls, bcls)


def _sc_gather(table, idx_all):
    info = plsc.get_sparse_core_info()
    nw = info.num_cores * info.num_subcores
    n = idx_all.shape[0]
    per_w = n // nw
    ch = 128
    n_ch = per_w // ch
    mesh = plsc.VectorSubcoreMesh(core_axis_name="c", subcore_axis_name="s")

    @functools.partial(
        pl.kernel, mesh=mesh,
        out_type=jax.ShapeDtypeStruct((n, D), jnp.float32),
        scratch_types=[
            pltpu.VMEM((ch,), jnp.int32),
            pltpu.VMEM((ch, D), jnp.float32),
            pltpu.SemaphoreType.DMA,
        ],
    )
    def k(table_hbm, idx_hbm, out_hbm, idx_v, rows_v, sem):
        wid = lax.axis_index("s") * info.num_cores + lax.axis_index("c")
        base = wid * per_w
        for c in range(n_ch):
            off = base + c * ch
            pltpu.sync_copy(idx_hbm.at[pl.ds(off, ch)], idx_v)
            pltpu.async_copy(table_hbm.at[idx_v], rows_v, sem).wait()
            pltpu.sync_copy(rows_v, out_hbm.at[pl.ds(off, ch)])

    return k(table, idx_all)


def kernel(input_ids, token_types, adm_index, age_ids, diag_code_group_dicts,
           task_id, token_emb, task_emb_table, W_occ, W_vis, W_next,
           a_o2v, a_next, W_cls, b_cls):
    f32 = jnp.float32
    b = input_ids.shape[0]
    v = age_ids.shape[1]

    age_pad = jnp.concatenate(
        [age_ids.astype(jnp.int32), jnp.zeros((b, VP - v), jnp.int32)], axis=1)
    idx_all = jnp.concatenate(
        [input_ids.reshape(-1).astype(jnp.int32), age_pad.reshape(-1)])
    rows = _sc_gather(token_emb, idx_all)
    se3 = rows[:b * L].reshape(b, L, D)
    vx3 = rows[b * L:].reshape(b, VP, D)

    ttr = token_types.astype(jnp.int32).reshape(b, 1, L)
    admr = adm_index.astype(jnp.int32).reshape(b, 1, L)
    ttc = token_types.astype(jnp.int32).reshape(b, L, 1)
    admc = adm_index.astype(jnp.int32).reshape(b, L, 1)

    eye = jnp.eye(H, dtype=f32)
    a0 = (a_o2v[0][:, :, None] * eye[:, None, :]).reshape(D, H)
    a1 = (a_o2v[1][:, :, None] * eye[:, None, :]).reshape(D, H)
    rmat = jnp.repeat(eye, DH, axis=1)
    tv = jnp.take(task_emb_table, jnp.asarray(task_id, jnp.int32),
                  axis=0).reshape(1, D)
    wcls = W_cls.reshape(1, D)
    bcls = b_cls.reshape(1, 1)

    out_p, log3 = _tc_call(se3, vx3, ttr, admr, ttc, admc,
                           W_occ, W_vis, W_next, a0, a1, rmat, tv, wcls, bcls)
    return log3.reshape(b), out_p[:, :v, :]


# speedup vs baseline: 16.9101x; 1.3447x over previous
import functools

import jax
import jax.numpy as jnp
from jax import lax
from jax.experimental import pallas as pl
from jax.experimental.pallas import tpu as pltpu
from jax.experimental.pallas import tpu_sc as plsc

L = 512
D = 256
H = 8
DH = D // H
VP = 64
AV = 64
NEG = -1e30
_dot = functools.partial(jnp.dot, preferred_element_type=jnp.float32,
                         precision=jax.lax.Precision.HIGHEST)


def _dot_bf(x, w):
    return jnp.dot(x.astype(jnp.bfloat16), w.astype(jnp.bfloat16),
                   preferred_element_type=jnp.float32)


def _tc_body(se_ref, vx_ref, ttr_ref, admr_ref,
             wocc_ref, wvis_ref, wnxt_ref, a0_ref, a1_ref, r_ref,
             tv_ref, wcls_ref, bcls_ref, out_ref, log_ref):
    f32 = jnp.float32
    se = se_ref[0]
    vx = vx_ref[0]
    ttr = ttr_ref[0]
    admr = admr_ref[0]

    keep_r = (ttr != 5) & (admr != 0)
    occ_r = keep_r & (ttr == 1)

    rows_a = lax.broadcasted_iota(jnp.int32, (AV, L), 0)
    oh_r = ((admr == rows_a) & keep_r).astype(f32)
    p_col = jnp.max(oh_r, axis=1, keepdims=True)
    lt_i = lax.broadcasted_iota(jnp.int32, (AV, AV), 0)
    lt_j = lax.broadcasted_iota(jnp.int32, (AV, AV), 1)
    ltri = (lt_j < lt_i).astype(f32)
    rank_col = _dot(ltri, p_col)
    nv = jnp.sum(p_col)
    dst_r = jnp.sum(oh_r * rank_col, axis=0, keepdims=True)

    rows_v = lax.broadcasted_iota(jnp.int32, (VP, L), 0).astype(f32)
    mf_b = (dst_r == rows_v) & occ_r
    mf = mf_b.astype(f32)
    mft = mf.T
    occ_cf = jnp.sum(mft, axis=1, keepdims=True)

    h_occ = _dot_bf(se, wocc_ref[...])
    h_vis = _dot_bf(vx, wvis_ref[...])
    h_nxt = _dot_bf(vx, wnxt_ref[...])

    e_src = _dot(h_occ, a0_ref[...])
    e_dst = _dot(h_vis, a1_ref[...])
    ge = _dot(mft, e_dst)
    e = e_src + ge
    e = jnp.where(e > 0, e, 0.2 * e)

    e_t = e.T
    m_cols = []
    for h in range(H):
        masked = jnp.where(mf_b, e_t[h:h + 1, :], NEG)
        m_h = jnp.max(masked, axis=1, keepdims=True)
        m_cols.append(jnp.where(m_h > -1e29, m_h, 0.0))
    m_vh = jnp.concatenate(m_cols, axis=1)
    m_used = _dot(mft, m_vh)

    ex = jnp.exp(e - m_used) * occ_cf
    den = _dot(mf, ex)
    rm = r_ref[...]
    ex_rep = _dot(ex, rm)
    num = _dot(mf, ex_rep * h_occ)
    den_rep = _dot(den, rm)
    agg1 = num / jnp.maximum(den_rep, 1e-9)

    s_i = lax.broadcasted_iota(jnp.int32, (VP, VP), 0)
    s_j = lax.broadcasted_iota(jnp.int32, (VP, VP), 1)
    shift = ((s_j == s_i - 1) & (s_i.astype(f32) < nv)).astype(f32)
    agg2 = _dot(shift, h_nxt)

    pre = agg1 + agg2 + h_vis
    out = jnp.where(pre > 0, pre, jnp.exp(pre) - 1.0)
    rowmask = lax.broadcasted_iota(jnp.int32, (VP, D), 0).astype(f32) < nv
    out_ref[0] = jnp.where(rowmask, out, 0.0)

    logit = jnp.sum(tv_ref[...] * wcls_ref[...], axis=1, keepdims=True)
    log_ref[0] = logit + bcls_ref[...]


def _tc_call(se3, vx3, ttr, admr, w_occ, w_vis, w_next,
             a0, a1, rmat, tv, wcls, bcls, interpret=False):
    b = se3.shape[0]
    f32 = jnp.float32
    fixed = lambda *s: pl.BlockSpec(s, lambda i: (0,) * len(s))
    per_b = lambda *s: pl.BlockSpec(s, lambda i: (i,) + (0,) * (len(s) - 1))
    return pl.pallas_call(
        _tc_body,
        grid=(b,),
        in_specs=[
            per_b(1, L, D), per_b(1, VP, D),
            per_b(1, 1, L), per_b(1, 1, L),
            fixed(D, D), fixed(D, D), fixed(D, D),
            fixed(D, H), fixed(D, H), fixed(H, D),
            fixed(1, D), fixed(1, D), fixed(1, 1),
        ],
        out_specs=[per_b(1, VP, D), per_b(1, 1, 1)],
        out_shape=[
            jax.ShapeDtypeStruct((b, VP, D), f32),
            jax.ShapeDtypeStruct((b, 1, 1), f32),
        ],
        compiler_params=pltpu.CompilerParams(
            dimension_semantics=("arbitrary",),
        ),
        interpret=interpret,
    )(se3, vx3, ttr, admr, w_occ, w_vis, w_next,
      a0, a1, rmat, tv, wcls, bcls)


def _sc_gather(table, idx_all):
    info = plsc.get_sparse_core_info()
    nw = info.num_cores * info.num_subcores
    n = idx_all.shape[0]
    per_w = n // nw
    ch = 128
    n_ch = per_w // ch
    mesh = plsc.VectorSubcoreMesh(core_axis_name="c", subcore_axis_name="s")

    @functools.partial(
        pl.kernel, mesh=mesh,
        out_type=jax.ShapeDtypeStruct((n, D), jnp.float32),
        scratch_types=[
            pltpu.VMEM((ch,), jnp.int32),
            pltpu.VMEM((ch, D), jnp.float32),
            pltpu.SemaphoreType.DMA,
        ],
    )
    def k(table_hbm, idx_hbm, out_hbm, idx_v, rows_v, sem):
        wid = lax.axis_index("s") * info.num_cores + lax.axis_index("c")
        base = wid * per_w
        for c in range(n_ch):
            off = base + c * ch
            pltpu.sync_copy(idx_hbm.at[pl.ds(off, ch)], idx_v)
            pltpu.async_copy(table_hbm.at[idx_v], rows_v, sem).wait()
            pltpu.sync_copy(rows_v, out_hbm.at[pl.ds(off, ch)])

    return k(table, idx_all)


def kernel(input_ids, token_types, adm_index, age_ids, diag_code_group_dicts,
           task_id, token_emb, task_emb_table, W_occ, W_vis, W_next,
           a_o2v, a_next, W_cls, b_cls):
    f32 = jnp.float32
    b = input_ids.shape[0]
    v = age_ids.shape[1]

    age_pad = jnp.concatenate(
        [age_ids.astype(jnp.int32), jnp.zeros((b, VP - v), jnp.int32)], axis=1)
    idx_all = jnp.concatenate(
        [input_ids.reshape(-1).astype(jnp.int32), age_pad.reshape(-1)])
    rows = _sc_gather(token_emb, idx_all)
    se3 = rows[:b * L].reshape(b, L, D)
    vx3 = rows[b * L:].reshape(b, VP, D)

    ttr = token_types.astype(jnp.int32).reshape(b, 1, L)
    admr = adm_index.astype(jnp.int32).reshape(b, 1, L)

    eye = jnp.eye(H, dtype=f32)
    a0 = (a_o2v[0][:, :, None] * eye[:, None, :]).reshape(D, H)
    a1 = (a_o2v[1][:, :, None] * eye[:, None, :]).reshape(D, H)
    rmat = jnp.repeat(eye, DH, axis=1)
    tv = jnp.take(task_emb_table, jnp.asarray(task_id, jnp.int32),
                  axis=0).reshape(1, D)
    wcls = W_cls.reshape(1, D)
    bcls = b_cls.reshape(1, 1)

    out_p, log3 = _tc_call(se3, vx3, ttr, admr,
                           W_occ, W_vis, W_next, a0, a1, rmat, tv, wcls, bcls)
    return log3.reshape(b), out_p[:, :v, :]


# speedup vs baseline: 30.6953x; 1.8152x over previous
import functools

import jax
import jax.numpy as jnp
from jax import lax
from jax.experimental import pallas as pl
from jax.experimental.pallas import tpu as pltpu
from jax.experimental.pallas import tpu_sc as plsc

L = 512
D = 256
H = 8
DH = D // H
VP = 64
V = 50
AV = 64
NEG = -1e30
_dot1 = functools.partial(jnp.dot, preferred_element_type=jnp.float32)


def _split_bf(v):
    v_hi = v.astype(jnp.bfloat16)
    v_lo = (v - v_hi.astype(jnp.float32)).astype(jnp.bfloat16)
    return v_hi, v_lo


def _dot_rsplit(m_bf, v):
    v_hi, v_lo = _split_bf(v)
    return _dot1(m_bf, v_hi) + _dot1(m_bf, v_lo)


def _dot_lsplit(v, m_bf):
    v_hi, v_lo = _split_bf(v)
    return _dot1(v_hi, m_bf) + _dot1(v_lo, m_bf)


def _tc_body(se_ref, vx_ref, ttr_ref, admr_ref,
             wocc_ref, wvis_ref, wnxt_ref, w0_ref, w1_ref, r_ref,
             tv_ref, wcls_ref, bcls_ref, out_ref, log_ref):
    f32 = jnp.float32
    bf16 = jnp.bfloat16
    se = se_ref[0].astype(bf16)
    vx = vx_ref[0].astype(bf16)
    ttr = ttr_ref[0]
    admr = admr_ref[0]

    keep_r = (ttr != 5) & (admr != 0)
    occ_r = keep_r & (ttr == 1)

    rows_a = lax.broadcasted_iota(jnp.int32, (AV, L), 0)
    oh_r = ((admr == rows_a) & keep_r).astype(f32)
    p_col = jnp.max(oh_r, axis=1, keepdims=True)
    lt_i = lax.broadcasted_iota(jnp.int32, (AV, AV), 0)
    lt_j = lax.broadcasted_iota(jnp.int32, (AV, AV), 1)
    ltri = (lt_j < lt_i).astype(bf16)
    rank_col = _dot1(ltri, p_col.astype(bf16))
    nv = jnp.sum(p_col)
    dst_r = jnp.sum(oh_r * rank_col, axis=0, keepdims=True)

    rows_v = lax.broadcasted_iota(jnp.int32, (VP, L), 0).astype(f32)
    mf_b = (dst_r == rows_v) & occ_r
    mf = mf_b.astype(f32)
    mf_bf = mf_b.astype(bf16)
    mft = mf.T
    mft_b = mft > 0.5
    mft_bf = mft.astype(bf16)
    occ_cf = jnp.sum(mft, axis=1, keepdims=True)

    h_occ = _dot1(se, wocc_ref[...])
    h_vis = _dot1(vx, wvis_ref[...])
    h_nxt = _dot1(vx, wnxt_ref[...])

    e_src = _dot1(se, w0_ref[...])
    e_dst = _dot1(vx, w1_ref[...])
    ge = _dot_rsplit(mft_bf, e_dst)
    e = e_src + ge
    e = jnp.where(e > 0, e, 0.2 * e)

    e_tokmax = jnp.max(e, axis=1, keepdims=True)
    masked = jnp.where(mft_b, e_tokmax, NEG)
    m_row = jnp.max(masked, axis=0, keepdims=True)
    m_row = jnp.where(m_row > -1e29, m_row, 0.0)
    m_used = jnp.sum(mft * m_row, axis=1, keepdims=True)

    ex = jnp.exp(e - m_used) * occ_cf
    den = _dot_rsplit(mf_bf, ex)
    rm = r_ref[...]
    ex_rep = _dot_lsplit(ex, rm)
    num = _dot_rsplit(mf_bf, ex_rep * h_occ)
    den_rep = _dot_lsplit(den, rm)
    agg1 = num / jnp.maximum(den_rep, 1e-9)

    s_i = lax.broadcasted_iota(jnp.int32, (VP, VP), 0)
    s_j = lax.broadcasted_iota(jnp.int32, (VP, VP), 1)
    shift = ((s_j == s_i - 1) & (s_i.astype(f32) < nv)).astype(bf16)
    agg2 = _dot_rsplit(shift, h_nxt)

    pre = agg1 + agg2 + h_vis
    out = jnp.where(pre > 0, pre, jnp.exp(pre) - 1.0)
    rowmask = lax.broadcasted_iota(jnp.int32, (VP, D), 0).astype(f32) < nv
    out_ref[0] = jnp.where(rowmask, out, 0.0)[:V]

    logit = jnp.sum(tv_ref[...] * wcls_ref[...], axis=1, keepdims=True)
    log_ref[0] = logit + bcls_ref[...]


def _tc_call(se3, vx3, ttr, admr, w_occ, w_vis, w_next,
             w0, w1, rmat, tv, wcls, bcls, interpret=False):
    b = se3.shape[0]
    f32 = jnp.float32
    fixed = lambda *s: pl.BlockSpec(s, lambda i: (0,) * len(s))
    per_b = lambda *s: pl.BlockSpec(s, lambda i: (i,) + (0,) * (len(s) - 1))
    return pl.pallas_call(
        _tc_body,
        grid=(b,),
        in_specs=[
            per_b(1, L, D), per_b(1, VP, D),
            per_b(1, 1, L), per_b(1, 1, L),
            fixed(D, D), fixed(D, D), fixed(D, D),
            fixed(D, H), fixed(D, H), fixed(H, D),
            fixed(1, D), fixed(1, D), fixed(1, 1),
        ],
        out_specs=[per_b(1, V, D), per_b(1, 1, 1)],
        out_shape=[
            jax.ShapeDtypeStruct((b, V, D), f32),
            jax.ShapeDtypeStruct((b, 1, 1), f32),
        ],
        compiler_params=pltpu.CompilerParams(
            dimension_semantics=("arbitrary",),
        ),
        interpret=interpret,
    )(se3, vx3, ttr, admr, w_occ, w_vis, w_next,
      w0, w1, rmat, tv, wcls, bcls)


def _sc_gather(table, idx_all, n_se, n_vx):
    info = plsc.get_sparse_core_info()
    nw = info.num_cores * info.num_subcores
    ch = 128
    se_ch = n_se // (nw * ch)
    vx_ch = n_vx // (nw * ch)
    mesh = plsc.VectorSubcoreMesh(core_axis_name="c", subcore_axis_name="s")

    @functools.partial(
        pl.kernel, mesh=mesh,
        out_type=[
            jax.ShapeDtypeStruct((n_se, D), jnp.float32),
            jax.ShapeDtypeStruct((n_vx, D), jnp.float32),
        ],
        scratch_types=[
            pltpu.VMEM((ch,), jnp.int32),
            pltpu.VMEM((ch, D), jnp.float32),
            pltpu.SemaphoreType.DMA,
        ],
    )
    def k(table_hbm, idx_hbm, out_se, out_vx, idx_v, rows_v, sem):
        wid = lax.axis_index("s") * info.num_cores + lax.axis_index("c")
        base_se = wid * (se_ch * ch)
        for c in range(se_ch):
            off = base_se + c * ch
            pltpu.sync_copy(idx_hbm.at[pl.ds(off, ch)], idx_v)
            pltpu.async_copy(table_hbm.at[idx_v], rows_v, sem).wait()
            pltpu.sync_copy(rows_v, out_se.at[pl.ds(off, ch)])
        base_vx = wid * (vx_ch * ch)
        for c in range(vx_ch):
            off = base_vx + c * ch
            pltpu.sync_copy(idx_hbm.at[pl.ds(n_se + off, ch)], idx_v)
            pltpu.async_copy(table_hbm.at[idx_v], rows_v, sem).wait()
            pltpu.sync_copy(rows_v, out_vx.at[pl.ds(off, ch)])

    return k(table, idx_all)


def kernel(input_ids, token_types, adm_index, age_ids, diag_code_group_dicts,
           task_id, token_emb, task_emb_table, W_occ, W_vis, W_next,
           a_o2v, a_next, W_cls, b_cls):
    f32 = jnp.float32
    bf16 = jnp.bfloat16
    b = input_ids.shape[0]
    v = age_ids.shape[1]

    age_pad = jnp.concatenate(
        [age_ids.astype(jnp.int32), jnp.zeros((b, VP - v), jnp.int32)], axis=1)
    idx_all = jnp.concatenate(
        [input_ids.reshape(-1).astype(jnp.int32), age_pad.reshape(-1)])
    se_flat, vx_flat = _sc_gather(token_emb, idx_all, b * L, b * VP)
    se3 = se_flat.reshape(b, L, D)
    vx3 = vx_flat.reshape(b, VP, D)

    ttr = token_types.astype(jnp.int32).reshape(b, 1, L)
    admr = adm_index.astype(jnp.int32).reshape(b, 1, L)

    eye = jnp.eye(H, dtype=f32)
    a0 = (a_o2v[0][:, :, None] * eye[:, None, :]).reshape(D, H)
    a1 = (a_o2v[1][:, :, None] * eye[:, None, :]).reshape(D, H)
    hp = jax.lax.Precision.HIGHEST
    w0 = jnp.dot(W_occ, a0, precision=hp).astype(bf16)
    w1 = jnp.dot(W_vis, a1, precision=hp).astype(bf16)
    rmat = jnp.repeat(eye, DH, axis=1).astype(bf16)
    tv = jnp.take(task_emb_table, jnp.asarray(task_id, jnp.int32),
                  axis=0).reshape(1, D)
    wcls = W_cls.reshape(1, D)
    bcls = b_cls.reshape(1, 1)

    out_p, log3 = _tc_call(se3, vx3, ttr, admr,
                           W_occ.astype(bf16), W_vis.astype(bf16),
                           W_next.astype(bf16), w0, w1, rmat, tv, wcls, bcls)
    return log3.reshape(b), out_p


# speedup vs baseline: 31.4782x; 1.0255x over previous
import functools

import jax
import jax.numpy as jnp
from jax import lax
from jax.experimental import pallas as pl
from jax.experimental.pallas import tpu as pltpu
from jax.experimental.pallas import tpu_sc as plsc

L = 512
D = 256
H = 8
DH = D // H
VP = 64
V = 50
AV = 64
NEG = -1e30
_dot1 = functools.partial(jnp.dot, preferred_element_type=jnp.float32)


def _split_bf(v):
    v_hi = v.astype(jnp.bfloat16)
    v_lo = (v - v_hi.astype(jnp.float32)).astype(jnp.bfloat16)
    return v_hi, v_lo


def _dot_rsplit(m_bf, v):
    v_hi, v_lo = _split_bf(v)
    return _dot1(m_bf, v_hi) + _dot1(m_bf, v_lo)


def _dot_lsplit(v, m_bf):
    v_hi, v_lo = _split_bf(v)
    return _dot1(v_hi, m_bf) + _dot1(v_lo, m_bf)


def _tc_body(se_ref, vx_ref, ttr_ref, admr_ref,
             wocc_ref, wvis_ref, wnxt_ref, w0_ref, w1_ref, r_ref,
             tv_ref, wcls_ref, bcls_ref, out_ref, log_ref):
    f32 = jnp.float32
    bf16 = jnp.bfloat16
    se = se_ref[0].astype(bf16)
    vx = vx_ref[0].astype(bf16)
    ttr = ttr_ref[0]
    admr = admr_ref[0]

    keep_r = (ttr != 5) & (admr != 0)
    occ_r = keep_r & (ttr == 1)

    rows_a = lax.broadcasted_iota(jnp.int32, (AV, L), 0)
    oh_r = ((admr == rows_a) & keep_r).astype(f32)
    p_col = jnp.max(oh_r, axis=1, keepdims=True)
    lt_i = lax.broadcasted_iota(jnp.int32, (AV, AV), 0)
    lt_j = lax.broadcasted_iota(jnp.int32, (AV, AV), 1)
    ltri = (lt_j < lt_i).astype(bf16)
    rank_col = _dot1(ltri, p_col.astype(bf16))
    nv = jnp.sum(p_col)
    dst_r = jnp.sum(oh_r * rank_col, axis=0, keepdims=True)

    rows_v = lax.broadcasted_iota(jnp.int32, (VP, L), 0).astype(f32)
    mf_b = (dst_r == rows_v) & occ_r
    mf = mf_b.astype(f32)
    mf_bf = mf_b.astype(bf16)
    mft = mf.T
    mft_b = mft > 0.5
    mft_bf = mft.astype(bf16)
    occ_cf = jnp.sum(mft, axis=1, keepdims=True)

    h_occ = _dot1(se, wocc_ref[...])
    h_vis = _dot1(vx, wvis_ref[...])
    h_nxt = _dot1(vx, wnxt_ref[...])

    e_src = _dot1(se, w0_ref[...])
    e_dst = _dot1(vx, w1_ref[...])
    ge = _dot_rsplit(mft_bf, e_dst)
    e = e_src + ge
    e = jnp.where(e > 0, e, 0.2 * e)

    e_tokmax = jnp.max(e, axis=1, keepdims=True)
    masked = jnp.where(mft_b, e_tokmax, NEG)
    m_row = jnp.max(masked, axis=0, keepdims=True)
    m_row = jnp.where(m_row > -1e29, m_row, 0.0)
    m_used = jnp.sum(mft * m_row, axis=1, keepdims=True)

    ex = jnp.exp(e - m_used) * occ_cf
    den = _dot_rsplit(mf_bf, ex)
    rm = r_ref[...]
    ex_rep = _dot_lsplit(ex, rm)
    num = _dot_rsplit(mf_bf, ex_rep * h_occ)
    den_rep = _dot_lsplit(den, rm)
    agg1 = num / jnp.maximum(den_rep, 1e-9)

    s_i = lax.broadcasted_iota(jnp.int32, (VP, VP), 0)
    s_j = lax.broadcasted_iota(jnp.int32, (VP, VP), 1)
    shift = ((s_j == s_i - 1) & (s_i.astype(f32) < nv)).astype(bf16)
    agg2 = _dot_rsplit(shift, h_nxt)

    pre = agg1 + agg2 + h_vis
    out = jnp.where(pre > 0, pre, jnp.exp(pre) - 1.0)
    rowmask = lax.broadcasted_iota(jnp.int32, (VP, D), 0).astype(f32) < nv
    out_ref[0] = jnp.where(rowmask, out, 0.0)[:V]

    logit = jnp.sum(tv_ref[...] * wcls_ref[...], axis=1, keepdims=True)
    log_ref[0] = logit + bcls_ref[...]


def _tc_call(se3, vx3, ttr, admr, w_occ, w_vis, w_next,
             w0, w1, rmat, tv, wcls, bcls, interpret=False):
    b = se3.shape[0]
    f32 = jnp.float32
    fixed = lambda *s: pl.BlockSpec(s, lambda i: (0,) * len(s))
    per_b = lambda *s: pl.BlockSpec(s, lambda i: (i,) + (0,) * (len(s) - 1))
    return pl.pallas_call(
        _tc_body,
        grid=(b,),
        in_specs=[
            per_b(1, L, D), per_b(1, VP, D),
            per_b(1, 1, L), per_b(1, 1, L),
            fixed(D, D), fixed(D, D), fixed(D, D),
            fixed(D, H), fixed(D, H), fixed(H, D),
            fixed(1, D), fixed(1, D), fixed(1, 1),
        ],
        out_specs=[per_b(1, V, D), per_b(1, 1, 1)],
        out_shape=[
            jax.ShapeDtypeStruct((b, V, D), f32),
            jax.ShapeDtypeStruct((b, 1, 1), f32),
        ],
        compiler_params=pltpu.CompilerParams(
            dimension_semantics=("arbitrary",),
        ),
        interpret=interpret,
    )(se3, vx3, ttr, admr, w_occ, w_vis, w_next,
      w0, w1, rmat, tv, wcls, bcls)


def _sc_gather(table, idx_all, n_se, n_vx):
    info = plsc.get_sparse_core_info()
    nw = info.num_cores * info.num_subcores
    ch = 128
    se_ch = n_se // (nw * ch)
    vx_ch = n_vx // (nw * ch)
    mesh = plsc.VectorSubcoreMesh(core_axis_name="c", subcore_axis_name="s")

    n_ch = se_ch + vx_ch
    per_w = n_ch * ch

    @functools.partial(
        pl.kernel, mesh=mesh,
        out_type=[
            jax.ShapeDtypeStruct((n_se, D), jnp.float32),
            jax.ShapeDtypeStruct((n_vx, D), jnp.float32),
        ],
        scratch_types=[
            pltpu.VMEM((per_w,), jnp.int32),
            pltpu.VMEM((2, ch, D), jnp.float32),
            pltpu.SemaphoreType.DMA((2,)),
            pltpu.SemaphoreType.DMA((2,)),
        ],
    )
    def k(table_hbm, idx_hbm, out_se, out_vx, idx_v, rows_v, gsem, wsem):
        wid = lax.axis_index("s") * info.num_cores + lax.axis_index("c")
        base_se = wid * (se_ch * ch)
        base_vx = wid * (vx_ch * ch)
        pltpu.sync_copy(idx_hbm.at[pl.ds(base_se, se_ch * ch)],
                        idx_v.at[pl.ds(0, se_ch * ch)])
        pltpu.sync_copy(idx_hbm.at[pl.ds(n_se + base_vx, vx_ch * ch)],
                        idx_v.at[pl.ds(se_ch * ch, vx_ch * ch)])

        def _gather(c):
            return pltpu.make_async_copy(
                table_hbm.at[idx_v.at[pl.ds(c * ch, ch)]],
                rows_v.at[c % 2], gsem.at[c % 2])

        def _write(c):
            if c < se_ch:
                dst = out_se.at[pl.ds(base_se + c * ch, ch)]
            else:
                dst = out_vx.at[pl.ds(base_vx + (c - se_ch) * ch, ch)]
            return pltpu.make_async_copy(rows_v.at[c % 2], dst, wsem.at[c % 2])

        _gather(0).start()
        for c in range(n_ch):
            if c + 1 < n_ch:
                if c >= 1:
                    _write(c - 1).wait()
                _gather(c + 1).start()
            _gather(c).wait()
            _write(c).start()
        if n_ch >= 2:
            _write(n_ch - 2).wait()
        _write(n_ch - 1).wait()

    return k(table, idx_all)


def kernel(input_ids, token_types, adm_index, age_ids, diag_code_group_dicts,
           task_id, token_emb, task_emb_table, W_occ, W_vis, W_next,
           a_o2v, a_next, W_cls, b_cls):
    f32 = jnp.float32
    bf16 = jnp.bfloat16
    b = input_ids.shape[0]
    v = age_ids.shape[1]

    age_pad = jnp.concatenate(
        [age_ids.astype(jnp.int32), jnp.zeros((b, VP - v), jnp.int32)], axis=1)
    idx_all = jnp.concatenate(
        [input_ids.reshape(-1).astype(jnp.int32), age_pad.reshape(-1)])
    se_flat, vx_flat = _sc_gather(token_emb, idx_all, b * L, b * VP)
    se3 = se_flat.reshape(b, L, D)
    vx3 = vx_flat.reshape(b, VP, D)

    ttr = token_types.astype(jnp.int32).reshape(b, 1, L)
    admr = adm_index.astype(jnp.int32).reshape(b, 1, L)

    eye = jnp.eye(H, dtype=f32)
    a0 = (a_o2v[0][:, :, None] * eye[:, None, :]).reshape(D, H)
    a1 = (a_o2v[1][:, :, None] * eye[:, None, :]).reshape(D, H)
    hp = jax.lax.Precision.HIGHEST
    w0 = jnp.dot(W_occ, a0, precision=hp).astype(bf16)
    w1 = jnp.dot(W_vis, a1, precision=hp).astype(bf16)
    rmat = jnp.repeat(eye, DH, axis=1).astype(bf16)
    tv = jnp.take(task_emb_table, jnp.asarray(task_id, jnp.int32),
                  axis=0).reshape(1, D)
    wcls = W_cls.reshape(1, D)
    bcls = b_cls.reshape(1, 1)

    out_p, log3 = _tc_call(se3, vx3, ttr, admr,
                           W_occ.astype(bf16), W_vis.astype(bf16),
                           W_next.astype(bf16), w0, w1, rmat, tv, wcls, bcls)
    return log3.reshape(b), out_p
